# baseline (speedup 1.0000x reference)
import functools

import jax
import jax.numpy as jnp
from jax import lax
from jax.experimental import pallas as pl
from jax.experimental.pallas import tpu as pltpu
from jax.experimental.pallas import tpu_sc as plsc

_NC = 2
_NS = 16
_H = 64
_NBUF = 3


def _sc_aggregate_body(e_rows, half, zrows, spt,
                       src2d, dst2d, table, zeros, lin2d, out,
                       idx_src, dst_raw, adj, rows, acc, lin,
                       sem_i, sem_g, sem_s):
    c = lax.axis_index("c")
    sid = lax.axis_index("s")
    base = c * half

    pltpu.sync_copy(zeros, acc.at[pl.ds(sid * zrows, zrows)])
    pltpu.sync_copy(lin2d, lin)
    plsc.subcore_barrier()

    r0 = sid * spt

    def istart(k, bk):
        row = jnp.minimum(r0 + k, e_rows - 1)
        pltpu.async_copy(src2d.at[pl.ds(row, 1)], idx_src.at[pl.ds(bk, 1)], sem_i)
        pltpu.async_copy(dst2d.at[pl.ds(row, 1)], dst_raw.at[pl.ds(bk, 1)], sem_i)

    def iwait(bk):
        pltpu.make_async_copy(src2d.at[pl.ds(0, 1)], idx_src.at[pl.ds(bk, 1)], sem_i).wait()
        pltpu.make_async_copy(dst2d.at[pl.ds(0, 1)], dst_raw.at[pl.ds(bk, 1)], sem_i).wait()

    def adj_compute(bk):
        for gi in range(8):
            d = dst_raw[bk, pl.ds(gi * 16, 16)]
            loc = d - base
            ok = (loc >= 0) & (loc < half)
            adj[bk, pl.ds(gi * 16, 16)] = jnp.where(
                ok, loc, half + sid * 16 + (d & 15))
            s = idx_src[bk, pl.ds(gi * 16, 16)]
            hot = lin[0, pl.ds(gi * 16, 16)] + sid * 128
            idx_src[bk, pl.ds(gi * 16, 16)] = jnp.where(ok, s, hot)

    def gstart(bk):
        pltpu.async_copy(table.at[idx_src.at[bk]], rows.at[bk], sem_g)

    def gwait(bk):
        pltpu.make_async_copy(table.at[pl.ds(0, 128)], rows.at[bk], sem_g).wait()

    def sstart(bk):
        pltpu.async_copy(rows.at[bk], acc.at[adj.at[bk]], sem_s, add=True)

    def swait():
        pltpu.make_async_copy(table.at[pl.ds(0, 128)], acc.at[pl.ds(0, 128)], sem_s).wait()

    istart(0, 0)
    iwait(0)
    adj_compute(0)
    gstart(0)
    istart(1, 1)

    def outer(g, carry):
        for b in range(_NBUF):
            s = g * _NBUF + b
            bn = (b + 1) % _NBUF
            bi = (b + 2) % _NBUF
            iwait(bn)
            if b == _NBUF - 1:
                swait()
            else:
                @pl.when(g >= 1)
                def _():
                    swait()
            adj_compute(bn)
            gstart(bn)
            istart(s + 2, bi)
            gwait(b)
            sstart(b)
        return carry

    lax.fori_loop(0, spt // _NBUF, outer, 0)

    swait()
    swait()
    gwait(0)
    iwait(1)
    plsc.subcore_barrier()

    big = (half + _NS - 1) // _NS
    big = ((big + 7) // 8) * 8
    last = half - (_NS - 1) * big

    @pl.when(sid < _NS - 1)
    def _():
        pltpu.sync_copy(acc.at[pl.ds(sid * big, big)],
                        out.at[pl.ds(base + sid * big, big)])

    @pl.when(sid == _NS - 1)
    def _():
        pltpu.sync_copy(acc.at[pl.ds((_NS - 1) * big, last)],
                        out.at[pl.ds(base + (_NS - 1) * big, last)])


def _sc_aggregate(dst_idx, src_idx, table, num_dst):
    e = dst_idx.shape[0]
    assert num_dst % _NC == 0
    half = num_dst // _NC
    acc_rows = half + 264
    assert acc_rows % _NS == 0
    zrows = acc_rows // _NS

    grain = _NS * 128 * _NBUF
    e_pad = ((e + grain - 1) // grain) * grain
    pad = e_pad - e
    if pad:
        src_idx = jnp.concatenate([src_idx, jnp.zeros((pad,), jnp.int32)])
        dst_idx = jnp.concatenate(
            [dst_idx, jnp.full((pad,), num_dst, jnp.int32)])
    src2d = src_idx.reshape(-1, 128)
    dst2d = dst_idx.reshape(-1, 128)
    e_rows = e_pad // 128
    spt = e_rows // _NS
    zeros = jnp.zeros((zrows, _H), jnp.float32)
    lin2d = jnp.arange(128, dtype=jnp.int32).reshape(1, 128)

    mesh = plsc.VectorSubcoreMesh(core_axis_name="c", subcore_axis_name="s",
                                  num_cores=_NC, num_subcores=_NS)
    body = functools.partial(_sc_aggregate_body, e_rows, half, zrows, spt)
    fn = pl.kernel(
        body,
        out_type=jax.ShapeDtypeStruct((num_dst, _H), jnp.float32),
        mesh=mesh,
        scratch_types=[
            pltpu.VMEM((_NBUF, 128), jnp.int32),
            pltpu.VMEM((_NBUF, 128), jnp.int32),
            pltpu.VMEM((_NBUF, 128), jnp.int32),
            pltpu.VMEM((_NBUF, 128, _H), jnp.float32),
            pltpu.VMEM_SHARED((acc_rows, _H), jnp.float32),
            pltpu.VMEM((1, 128), jnp.int32),
            pltpu.SemaphoreType.DMA,
            pltpu.SemaphoreType.DMA,
            pltpu.SemaphoreType.DMA,
        ],
        compiler_params=pltpu.CompilerParams(use_tc_tiling_on_sc=False),
    )
    return fn(src2d, dst2d, table, zeros, lin2d)


_BLK = 1000


def _update_block(h_ref, aggr_ref, w1t_ref, b1_ref, w2at_ref, w2bt_ref, b2_ref, out_ref):
    aggr = aggr_ref[...]
    msg = jnp.maximum(
        jnp.dot(aggr, w1t_ref[...], preferred_element_type=jnp.float32) + b1_ref[...],
        0.0,
    )
    out = (
        jnp.dot(h_ref[...], w2at_ref[...], preferred_element_type=jnp.float32)
        + jnp.dot(msg, w2bt_ref[...], preferred_element_type=jnp.float32)
        + b2_ref[...]
    )
    n = jnp.sqrt(jnp.sum(out * out, axis=1, keepdims=True))
    out_ref[...] = out / jnp.maximum(n, 1e-12)


def _dense_update(h, aggr, W1, b1, W2, b2):
    n, hdim = h.shape
    assert n % _BLK == 0
    grid = (n // _BLK,)
    w1t = W1.T
    w2at = W2[:, :hdim].T
    w2bt = W2[:, hdim:].T
    b1r = b1.reshape(1, hdim)
    b2r = b2.reshape(1, hdim)
    row_spec = pl.BlockSpec((_BLK, hdim), lambda i: (i, 0))
    full_spec = pl.BlockSpec((hdim, hdim), lambda i: (0, 0))
    bias_spec = pl.BlockSpec((1, hdim), lambda i: (0, 0))
    return pl.pallas_call(
        _update_block,
        grid=grid,
        in_specs=[row_spec, row_spec, full_spec, bias_spec, full_spec, full_spec, bias_spec],
        out_specs=row_spec,
        out_shape=jax.ShapeDtypeStruct((n, hdim), jnp.float32),
    )(h, aggr, w1t, b1r, w2at, w2bt, b2r)


def kernel(user_song_adj, song_artist_adj, user_emb, song_emb, artist_emb,
           W_as, b_as, W_s, b_s, W_su, b_su, W_u, b_u):
    num_users = user_emb.shape[0]
    num_songs = song_emb.shape[0]
    aggr_artist = _sc_aggregate(song_artist_adj[0], song_artist_adj[1],
                                artist_emb, num_songs)
    h_s_new = _dense_update(song_emb, aggr_artist, W_as, b_as, W_s, b_s)
    aggr_song = _sc_aggregate(user_song_adj[0], user_song_adj[1],
                              h_s_new, num_users)
    h_u_new = _dense_update(user_emb, aggr_song, W_su, b_su, W_u, b_u)
    return (h_u_new, h_s_new)


# speedup vs baseline: 1.0649x; 1.0649x over previous
import functools

import jax
import jax.numpy as jnp
from jax import lax
from jax.experimental import pallas as pl
from jax.experimental.pallas import tpu as pltpu
from jax.experimental.pallas import tpu_sc as plsc

_NC = 2
_NS = 16
_H = 64
_NBUF = 3


def _sc_aggregate_body(e_rows, half, zrows, spt,
                       src2d, dst2d, table, zeros, lin2d, out,
                       idx_src, dst_raw, adj, rows, acc, lin,
                       sem_i, sem_g, sem_s):
    c = lax.axis_index("c")
    sid = lax.axis_index("s")
    base = c * half

    pltpu.sync_copy(zeros, acc.at[pl.ds(sid * zrows, zrows)])
    pltpu.sync_copy(lin2d, lin)
    plsc.subcore_barrier()

    r0 = sid * spt

    def istart(k, bk):
        row = jnp.minimum(r0 + k, e_rows - 1)
        pltpu.async_copy(src2d.at[pl.ds(row, 1)], idx_src.at[pl.ds(bk, 1)], sem_i)
        pltpu.async_copy(dst2d.at[pl.ds(row, 1)], dst_raw.at[pl.ds(bk, 1)], sem_i)

    def iwait(bk):
        pltpu.make_async_copy(src2d.at[pl.ds(0, 1)], idx_src.at[pl.ds(bk, 1)], sem_i).wait()
        pltpu.make_async_copy(dst2d.at[pl.ds(0, 1)], dst_raw.at[pl.ds(bk, 1)], sem_i).wait()

    def adj_compute(bk):
        for gi in range(8):
            d = dst_raw[bk, pl.ds(gi * 16, 16)]
            loc = d - base
            ok = (loc >= 0) & (loc < half)
            adj[bk, pl.ds(gi * 16, 16)] = jnp.where(
                ok, loc, half + sid * 16 + (d & 15))
            s = idx_src[bk, pl.ds(gi * 16, 16)]
            hot = lin[0, pl.ds(gi * 16, 16)] + sid * 128
            idx_src[bk, pl.ds(gi * 16, 16)] = jnp.where(ok, s, hot)

    def gstart(bk):
        pltpu.async_copy(table.at[idx_src.at[bk]], rows.at[bk], sem_g)

    def gwait(bk):
        pltpu.make_async_copy(table.at[pl.ds(0, 128)], rows.at[bk], sem_g).wait()

    def sstart(bk):
        pltpu.async_copy(rows.at[bk], acc.at[adj.at[bk]], sem_s, add=True)

    def swait():
        pltpu.make_async_copy(table.at[pl.ds(0, 128)], acc.at[pl.ds(0, 128)], sem_s).wait()

    istart(0, 0)
    iwait(0)
    adj_compute(0)
    gstart(0)
    istart(1, 1)

    def outer(g, carry):
        for b in range(_NBUF):
            s = g * _NBUF + b
            bn = (b + 1) % _NBUF
            bi = (b + 2) % _NBUF
            iwait(bn)
            if b == _NBUF - 1:
                swait()
            else:
                @pl.when(g >= 1)
                def _():
                    swait()
            adj_compute(bn)
            gstart(bn)
            istart(s + 2, bi)
            gwait(b)
            sstart(b)
        return carry

    lax.fori_loop(0, spt // _NBUF, outer, 0)

    swait()
    swait()
    gwait(0)
    iwait(1)
    plsc.subcore_barrier()

    big = (half + _NS - 1) // _NS
    big = ((big + 7) // 8) * 8
    last = half - (_NS - 1) * big

    @pl.when(sid < _NS - 1)
    def _():
        pltpu.sync_copy(acc.at[pl.ds(sid * big, big)],
                        out.at[pl.ds(base + sid * big, big)])

    @pl.when(sid == _NS - 1)
    def _():
        pltpu.sync_copy(acc.at[pl.ds((_NS - 1) * big, last)],
                        out.at[pl.ds(base + (_NS - 1) * big, last)])


def _sc_aggregate(dst_idx, src_idx, table, num_dst):
    e = dst_idx.shape[0]
    assert num_dst % _NC == 0
    half = num_dst // _NC
    acc_rows = half + 264
    assert acc_rows % _NS == 0
    zrows = acc_rows // _NS

    grain = _NS * 128 * _NBUF
    e_pad = ((e + grain - 1) // grain) * grain
    pad = e_pad - e
    if pad:
        src_idx = jnp.concatenate([src_idx, jnp.zeros((pad,), jnp.int32)])
        dst_idx = jnp.concatenate(
            [dst_idx, jnp.full((pad,), num_dst, jnp.int32)])
    src2d = src_idx.reshape(-1, 128)
    dst2d = dst_idx.reshape(-1, 128)
    e_rows = e_pad // 128
    spt = e_rows // _NS
    zeros = jnp.zeros((zrows, _H), jnp.float32)
    lin2d = jnp.arange(128, dtype=jnp.int32).reshape(1, 128)

    mesh = plsc.VectorSubcoreMesh(core_axis_name="c", subcore_axis_name="s",
                                  num_cores=_NC, num_subcores=_NS)
    body = functools.partial(_sc_aggregate_body, e_rows, half, zrows, spt)
    fn = pl.kernel(
        body,
        out_type=jax.ShapeDtypeStruct((num_dst, _H), jnp.float32),
        mesh=mesh,
        scratch_types=[
            pltpu.VMEM((_NBUF, 128), jnp.int32),
            pltpu.VMEM((_NBUF, 128), jnp.int32),
            pltpu.VMEM((_NBUF, 128), jnp.int32),
            pltpu.VMEM((_NBUF, 128, _H), jnp.float32),
            pltpu.VMEM_SHARED((acc_rows, _H), jnp.float32),
            pltpu.VMEM((1, 128), jnp.int32),
            pltpu.SemaphoreType.DMA,
            pltpu.SemaphoreType.DMA,
            pltpu.SemaphoreType.DMA,
        ],
        compiler_params=pltpu.CompilerParams(use_tc_tiling_on_sc=False),
    )
    return fn(src2d, dst2d, table, zeros, lin2d)


_BLK = 5000


def _update_block(h_ref, aggr_ref, w1t_ref, b1_ref, w2at_ref, w2bt_ref, b2_ref, out_ref):
    aggr = aggr_ref[...]
    msg = jnp.maximum(
        jnp.dot(aggr, w1t_ref[...], preferred_element_type=jnp.float32) + b1_ref[...],
        0.0,
    )
    out = (
        jnp.dot(h_ref[...], w2at_ref[...], preferred_element_type=jnp.float32)
        + jnp.dot(msg, w2bt_ref[...], preferred_element_type=jnp.float32)
        + b2_ref[...]
    )
    n = jnp.sqrt(jnp.sum(out * out, axis=1, keepdims=True))
    out_ref[...] = out / jnp.maximum(n, 1e-12)


def _dense_update(h, aggr, W1, b1, W2, b2):
    n, hdim = h.shape
    assert n % _BLK == 0
    grid = (n // _BLK,)
    w1t = W1.T
    w2at = W2[:, :hdim].T
    w2bt = W2[:, hdim:].T
    b1r = b1.reshape(1, hdim)
    b2r = b2.reshape(1, hdim)
    row_spec = pl.BlockSpec((_BLK, hdim), lambda i: (i, 0))
    full_spec = pl.BlockSpec((hdim, hdim), lambda i: (0, 0))
    bias_spec = pl.BlockSpec((1, hdim), lambda i: (0, 0))
    return pl.pallas_call(
        _update_block,
        grid=grid,
        in_specs=[row_spec, row_spec, full_spec, bias_spec, full_spec, full_spec, bias_spec],
        out_specs=row_spec,
        out_shape=jax.ShapeDtypeStruct((n, hdim), jnp.float32),
    )(h, aggr, w1t, b1r, w2at, w2bt, b2r)


def kernel(user_song_adj, song_artist_adj, user_emb, song_emb, artist_emb,
           W_as, b_as, W_s, b_s, W_su, b_su, W_u, b_u):
    num_users = user_emb.shape[0]
    num_songs = song_emb.shape[0]
    aggr_artist = _sc_aggregate(song_artist_adj[0], song_artist_adj[1],
                                artist_emb, num_songs)
    h_s_new = _dense_update(song_emb, aggr_artist, W_as, b_as, W_s, b_s)
    aggr_song = _sc_aggregate(user_song_adj[0], user_song_adj[1],
                              h_s_new, num_users)
    h_u_new = _dense_update(user_emb, aggr_song, W_su, b_su, W_u, b_u)
    return (h_u_new, h_s_new)
